# baseline (speedup 1.0000x reference)
import functools

import jax
import jax.numpy as jnp
from jax import lax
from jax.experimental import pallas as pl
from jax.experimental.pallas import tpu as pltpu
from jax.experimental.pallas import tpu_sc as plsc

_LANES = 16


def _sc_gather(table, flat_idx):
    n_table = table.shape[0]
    n_idx = flat_idx.shape[0]
    info = plsc.get_sparse_core_info()
    n_workers = info.num_cores * info.num_subcores
    per = n_idx // n_workers
    steps = per // _LANES
    mesh = plsc.VectorSubcoreMesh(core_axis_name="c", subcore_axis_name="s")

    @functools.partial(
        pl.kernel,
        mesh=mesh,
        compiler_params=pltpu.CompilerParams(needs_layout_passes=False),
        out_type=jax.ShapeDtypeStruct((n_idx,), jnp.float32),
        scratch_types=[
            pltpu.VMEM((n_table,), jnp.float32),
            pltpu.VMEM((per,), jnp.int32),
            pltpu.VMEM((per,), jnp.float32),
        ],
    )
    def gather_kernel(deg_hbm, idx_hbm, out_hbm, table_v, idx_v, out_v):
        wid = lax.axis_index("s") * info.num_cores + lax.axis_index("c")
        base = wid * per
        pltpu.sync_copy(deg_hbm, table_v)
        pltpu.sync_copy(idx_hbm.at[pl.ds(base, per)], idx_v)

        def body(i, carry):
            off = i * _LANES
            idx = idx_v[pl.ds(off, _LANES)]
            out_v[pl.ds(off, _LANES)] = plsc.load_gather(table_v, [idx])
            return carry

        lax.fori_loop(0, steps, body, 0)
        pltpu.sync_copy(out_v, out_hbm.at[pl.ds(base, per)])

    return gather_kernel(table, flat_idx)


def _rank_select_body(w_row, g_row, w_col, g_col, ids_col, out_ref):
    m = w_col.shape[1]
    kp = out_ref.shape[2]
    eps = jnp.float32(1e-30)
    s_row = jnp.log(w_row[0] + eps) + g_row[0]
    s_col = jnp.log(w_col[0] + eps) + g_col[0]
    ii = lax.broadcasted_iota(jnp.int32, (m, 1), 0)

    chunk = 512
    rank = jnp.zeros((m, 1), jnp.int32)
    for jb in range(0, m, chunk):
        width = min(chunk, m - jb)
        sj = s_row[:, jb:jb + width]
        jj = jb + lax.broadcasted_iota(jnp.int32, (1, width), 1)
        beats = (sj > s_col) | ((sj == s_col) & (jj < ii))
        rank = rank + jnp.sum(beats.astype(jnp.int32), axis=1, keepdims=True)

    rr = lax.broadcasted_iota(jnp.int32, (1, kp), 1)
    sel = jnp.sum(jnp.where(rank == rr, ids_col[0], 0), axis=0, keepdims=True)
    out_ref[...] = sel[None]


def _tc_rank_select(w16, gumbel, ids):
    c, m = w16.shape
    kp = 512

    grid_spec = pl.GridSpec(
        grid=(c,),
        in_specs=[
            pl.BlockSpec((1, 1, m), lambda i: (i, 0, 0)),
            pl.BlockSpec((1, 1, m), lambda i: (i, 0, 0)),
            pl.BlockSpec((1, m, 1), lambda i: (i, 0, 0)),
            pl.BlockSpec((1, m, 1), lambda i: (i, 0, 0)),
            pl.BlockSpec((1, m, 1), lambda i: (i, 0, 0)),
        ],
        out_specs=pl.BlockSpec((1, 1, kp), lambda i: (i, 0, 0)),
    )

    def body(wr_ref, gr_ref, wc_ref, gc_ref, ic_ref, out_ref):
        _rank_select_body(
            wr_ref[...], gr_ref[...], wc_ref[...], gc_ref[...], ic_ref[...],
            out_ref)

    out = pl.pallas_call(
        body,
        grid_spec=grid_spec,
        out_shape=jax.ShapeDtypeStruct((c, 1, kp), jnp.int32),
    )(
        w16.reshape(c, 1, m),
        gumbel.reshape(c, 1, m),
        w16.reshape(c, m, 1),
        gumbel.reshape(c, m, 1),
        ids.reshape(c, m, 1),
    )
    return out.reshape(c, kp)


def kernel(in_degrees, ids_per_cls, budget):
    c, m = ids_per_cls.shape
    k = min(500, m)
    ids = ids_per_cls.astype(jnp.int32)

    gumbel = jax.random.gumbel(jax.random.key(42), (c, m), dtype=jnp.float32)

    n = c * m
    n_pad = -(-n // 512) * 512
    flat_ids = jnp.zeros((n_pad,), jnp.int32).at[:n].set(ids.reshape(-1))
    w = _sc_gather(in_degrees.astype(jnp.float32), flat_ids)[:n].reshape(c, m)
    w16 = w.astype(jnp.float16).astype(jnp.float32)

    sel = _tc_rank_select(w16, gumbel, ids)
    return sel[:, :k].reshape(-1).astype(ids_per_cls.dtype)


# speedup vs baseline: 4.0791x; 4.0791x over previous
import functools

import jax
import jax.numpy as jnp
from jax import lax
from jax.experimental import pallas as pl
from jax.experimental.pallas import tpu as pltpu
from jax.experimental.pallas import tpu_sc as plsc

_LANES = 16
_TARGET = 512
_CAND = 576


def _sc_gather(table, flat_idx):
    n_table = table.shape[0]
    n_idx = flat_idx.shape[0]
    info = plsc.get_sparse_core_info()
    n_workers = info.num_cores * info.num_subcores
    per = n_idx // n_workers
    steps = per // _LANES
    mesh = plsc.VectorSubcoreMesh(core_axis_name="c", subcore_axis_name="s")

    @functools.partial(
        pl.kernel,
        mesh=mesh,
        compiler_params=pltpu.CompilerParams(needs_layout_passes=False),
        out_type=jax.ShapeDtypeStruct((n_idx,), jnp.float32),
        scratch_types=[
            pltpu.VMEM((n_table,), jnp.float32),
            pltpu.VMEM((per,), jnp.int32),
            pltpu.VMEM((per,), jnp.float32),
        ],
    )
    def gather_kernel(deg_hbm, idx_hbm, out_hbm, table_v, idx_v, out_v):
        wid = lax.axis_index("s") * info.num_cores + lax.axis_index("c")
        base = wid * per
        pltpu.sync_copy(deg_hbm, table_v)
        pltpu.sync_copy(idx_hbm.at[pl.ds(base, per)], idx_v)

        def body(i, carry):
            off = i * _LANES
            idx = idx_v[pl.ds(off, _LANES)]
            out_v[pl.ds(off, _LANES)] = plsc.load_gather(table_v, [idx])
            return carry

        lax.fori_loop(0, steps, body, 0)
        pltpu.sync_copy(out_v, out_hbm.at[pl.ds(base, per)])

    return gather_kernel(table, flat_idx)


def _tc_keys_threshold(w16, gumbel):
    c, m = w16.shape

    def body(w_ref, g_ref, u_ref, b_ref):
        s = jnp.log(w_ref[...] + jnp.float32(1e-30)) + g_ref[...]
        b = lax.bitcast_convert_type(s, jnp.uint32)
        neg = (b >> jnp.uint32(31)) == jnp.uint32(1)
        u = b ^ jnp.where(neg, jnp.uint32(0xFFFFFFFF), jnp.uint32(0x80000000))
        u_ref[...] = u

        def bit_step(i, acc):
            bit = jnp.uint32(1) << (jnp.uint32(31) - i.astype(jnp.uint32))
            cand = acc | bit
            cnt = jnp.sum((u >= cand).astype(jnp.int32), axis=1,
                          keepdims=True)
            return jnp.where(cnt >= _TARGET, cand, acc)

        bsel = lax.fori_loop(0, 32, bit_step, jnp.zeros((c, 1), jnp.uint32))
        b_ref[...] = jnp.broadcast_to(bsel, (c, _LANES))

    return pl.pallas_call(
        body,
        out_shape=(
            jax.ShapeDtypeStruct((c, m), jnp.uint32),
            jax.ShapeDtypeStruct((c, _LANES), jnp.uint32),
        ),
    )(w16, gumbel)


def _sc_compact(u_keys, ids, thresh):
    c, m = u_keys.shape
    chunks = m // _LANES
    init_steps = _CAND // _LANES
    info = plsc.get_sparse_core_info()
    n_workers = info.num_cores * info.num_subcores
    n_rounds = -(-c // n_workers)
    mesh = plsc.VectorSubcoreMesh(core_axis_name="c", subcore_axis_name="s")

    @functools.partial(
        pl.kernel,
        mesh=mesh,
        compiler_params=pltpu.CompilerParams(needs_layout_passes=False),
        out_type=(
            jax.ShapeDtypeStruct((c, _CAND), jnp.uint32),
            jax.ShapeDtypeStruct((c, _CAND), jnp.int32),
            jax.ShapeDtypeStruct((c, _CAND), jnp.int32),
        ),
        scratch_types=[
            pltpu.VMEM((m,), jnp.uint32),
            pltpu.VMEM((m,), jnp.int32),
            pltpu.VMEM((_LANES,), jnp.uint32),
            pltpu.VMEM((_CAND,), jnp.uint32),
            pltpu.VMEM((_CAND,), jnp.int32),
            pltpu.VMEM((_CAND,), jnp.int32),
        ],
    )
    def compact_kernel(u_hbm, ids_hbm, b_hbm, cu_hbm, cids_hbm, cpos_hbm,
                       u_v, ids_v, b_v, cu_v, cids_v, cpos_v):
        wid = lax.axis_index("s") * info.num_cores + lax.axis_index("c")

        for rnd in range(n_rounds):
            cls = wid + rnd * n_workers

            @pl.when(cls < c)
            def _process():
                pltpu.sync_copy(u_hbm.at[cls], u_v)
                pltpu.sync_copy(ids_hbm.at[cls], ids_v)
                pltpu.sync_copy(b_hbm.at[cls], b_v)
                b_vec = b_v[...]

                def init(i, carry):
                    sl = pl.ds(i * _LANES, _LANES)
                    cu_v[sl] = jnp.zeros((_LANES,), jnp.uint32)
                    cids_v[sl] = jnp.zeros((_LANES,), jnp.int32)
                    cpos_v[sl] = jnp.full((_LANES,), 4095, jnp.int32)
                    return carry

                lax.fori_loop(0, init_steps, init, 0)

                def step(i, off):
                    sl = pl.ds(i * _LANES, _LANES)
                    kv = u_v[sl]
                    mask = kv >= b_vec

                    @pl.when(off <= _CAND - _LANES)
                    def _store():
                        dst = pl.ds(off, _LANES)
                        pos = i * _LANES + lax.iota(jnp.int32, _LANES)
                        plsc.store_compressed(cu_v.at[dst], kv, mask=mask)
                        plsc.store_compressed(cids_v.at[dst], ids_v[sl],
                                              mask=mask)
                        plsc.store_compressed(cpos_v.at[dst], pos, mask=mask)

                    return off + jnp.sum(mask.astype(jnp.int32))

                lax.fori_loop(0, chunks, step, jnp.int32(0))
                pltpu.sync_copy(cu_v, cu_hbm.at[cls])
                pltpu.sync_copy(cids_v, cids_hbm.at[cls])
                pltpu.sync_copy(cpos_v, cpos_hbm.at[cls])

    return compact_kernel(u_keys, ids, thresh)


def _tc_rank_emit(cu, cids, cpos):
    c = cu.shape[0]
    n = _CAND
    kp = _TARGET

    def body(ur_ref, pr_ref, uc_ref, pc_ref, ic_ref, out_ref):
        u_row = ur_ref[0]
        p_row = pr_ref[0]
        u_col = uc_ref[0]
        p_col = pc_ref[0]
        beats = (u_row > u_col) | ((u_row == u_col) & (p_row < p_col))
        rank = jnp.sum(beats.astype(jnp.int32), axis=1, keepdims=True)
        rr = lax.broadcasted_iota(jnp.int32, (1, kp), 1)
        sel = jnp.sum(jnp.where(rank == rr, ic_ref[0], 0), axis=0,
                      keepdims=True)
        out_ref[...] = sel[None]

    out = pl.pallas_call(
        body,
        grid=(c,),
        in_specs=[
            pl.BlockSpec((1, 1, n), lambda i: (i, 0, 0)),
            pl.BlockSpec((1, 1, n), lambda i: (i, 0, 0)),
            pl.BlockSpec((1, n, 1), lambda i: (i, 0, 0)),
            pl.BlockSpec((1, n, 1), lambda i: (i, 0, 0)),
            pl.BlockSpec((1, n, 1), lambda i: (i, 0, 0)),
        ],
        out_specs=pl.BlockSpec((1, 1, kp), lambda i: (i, 0, 0)),
        out_shape=jax.ShapeDtypeStruct((c, 1, kp), jnp.int32),
    )(
        cu.reshape(c, 1, n),
        cpos.reshape(c, 1, n),
        cu.reshape(c, n, 1),
        cpos.reshape(c, n, 1),
        cids.reshape(c, n, 1),
    )
    return out.reshape(c, kp)


def kernel(in_degrees, ids_per_cls, budget):
    c, m = ids_per_cls.shape
    k = min(500, m)
    ids = ids_per_cls.astype(jnp.int32)

    gumbel = jax.random.gumbel(jax.random.key(42), (c, m), dtype=jnp.float32)

    n = c * m
    n_pad = -(-n // 512) * 512
    flat_ids = jnp.zeros((n_pad,), jnp.int32).at[:n].set(ids.reshape(-1))
    w = _sc_gather(in_degrees.astype(jnp.float32), flat_ids)[:n].reshape(c, m)
    w16 = w.astype(jnp.float16).astype(jnp.float32)

    u_keys, thresh = _tc_keys_threshold(w16, gumbel)

    cu, cids, cpos = _sc_compact(u_keys, ids, thresh)

    sel = _tc_rank_emit(cu, cids, cpos)
    return sel[:, :k].reshape(-1).astype(ids_per_cls.dtype)


# speedup vs baseline: 4.1436x; 1.0158x over previous
import functools

import jax
import jax.numpy as jnp
from jax import lax
from jax.experimental import pallas as pl
from jax.experimental.pallas import tpu as pltpu
from jax.experimental.pallas import tpu_sc as plsc

_LANES = 16
_TARGET = 500
_CAND = 528


def _sc_gather(table, flat_idx):
    n_table = table.shape[0]
    n_idx = flat_idx.shape[0]
    info = plsc.get_sparse_core_info()
    n_workers = info.num_cores * info.num_subcores
    per = n_idx // n_workers
    steps = per // _LANES
    mesh = plsc.VectorSubcoreMesh(core_axis_name="c", subcore_axis_name="s")

    @functools.partial(
        pl.kernel,
        mesh=mesh,
        compiler_params=pltpu.CompilerParams(needs_layout_passes=False),
        out_type=jax.ShapeDtypeStruct((n_idx,), jnp.float32),
        scratch_types=[
            pltpu.VMEM((n_table,), jnp.float32),
            pltpu.VMEM((per,), jnp.int32),
            pltpu.VMEM((per,), jnp.float32),
        ],
    )
    def gather_kernel(deg_hbm, idx_hbm, out_hbm, table_v, idx_v, out_v):
        wid = lax.axis_index("s") * info.num_cores + lax.axis_index("c")
        base = wid * per
        pltpu.sync_copy(deg_hbm, table_v)
        pltpu.sync_copy(idx_hbm.at[pl.ds(base, per)], idx_v)

        def body(i, carry):
            off = i * _LANES
            idx = idx_v[pl.ds(off, _LANES)]
            out_v[pl.ds(off, _LANES)] = plsc.load_gather(table_v, [idx])
            return carry

        lax.fori_loop(0, steps, body, 0)
        pltpu.sync_copy(out_v, out_hbm.at[pl.ds(base, per)])

    return gather_kernel(table, flat_idx)


def _tc_keys_threshold(w16, gumbel):
    c, m = w16.shape

    def body(w_ref, g_ref, u_ref, b_ref):
        s = jnp.log(w_ref[...] + jnp.float32(1e-30)) + g_ref[...]
        b = lax.bitcast_convert_type(s, jnp.uint32)
        neg = (b >> jnp.uint32(31)) == jnp.uint32(1)
        u = b ^ jnp.where(neg, jnp.uint32(0xFFFFFFFF), jnp.uint32(0x80000000))
        u_ref[...] = u

        def bit_step(i, acc):
            bit = jnp.uint32(1) << (jnp.uint32(31) - i.astype(jnp.uint32))
            cand = acc | bit
            cnt = jnp.sum((u >= cand).astype(jnp.int32), axis=1,
                          keepdims=True)
            return jnp.where(cnt >= _TARGET, cand, acc)

        bsel = lax.fori_loop(0, 32, bit_step, jnp.zeros((c, 1), jnp.uint32))
        b_ref[...] = jnp.broadcast_to(bsel, (c, _LANES))

    return pl.pallas_call(
        body,
        out_shape=(
            jax.ShapeDtypeStruct((c, m), jnp.uint32),
            jax.ShapeDtypeStruct((c, _LANES), jnp.uint32),
        ),
    )(w16, gumbel)


def _sc_compact(u_keys, ids, thresh):
    c, m = u_keys.shape
    chunks = m // _LANES
    init_steps = _CAND // _LANES
    info = plsc.get_sparse_core_info()
    n_workers = info.num_cores * info.num_subcores
    n_rounds = -(-c // n_workers)
    mesh = plsc.VectorSubcoreMesh(core_axis_name="c", subcore_axis_name="s")

    @functools.partial(
        pl.kernel,
        mesh=mesh,
        compiler_params=pltpu.CompilerParams(needs_layout_passes=False),
        out_type=(
            jax.ShapeDtypeStruct((c, _CAND), jnp.uint32),
            jax.ShapeDtypeStruct((c, _CAND), jnp.int32),
            jax.ShapeDtypeStruct((c, _CAND), jnp.int32),
        ),
        scratch_types=[
            pltpu.VMEM((m,), jnp.uint32),
            pltpu.VMEM((m,), jnp.int32),
            pltpu.VMEM((_LANES,), jnp.uint32),
            pltpu.VMEM((_CAND,), jnp.uint32),
            pltpu.VMEM((_CAND,), jnp.int32),
            pltpu.VMEM((_CAND,), jnp.int32),
        ],
    )
    def compact_kernel(u_hbm, ids_hbm, b_hbm, cu_hbm, cids_hbm, cpos_hbm,
                       u_v, ids_v, b_v, cu_v, cids_v, cpos_v):
        wid = lax.axis_index("s") * info.num_cores + lax.axis_index("c")

        for rnd in range(n_rounds):
            cls = wid + rnd * n_workers

            @pl.when(cls < c)
            def _process():
                pltpu.sync_copy(u_hbm.at[cls], u_v)
                pltpu.sync_copy(ids_hbm.at[cls], ids_v)
                pltpu.sync_copy(b_hbm.at[cls], b_v)
                b_vec = b_v[...]

                def init(i, carry):
                    sl = pl.ds(i * _LANES, _LANES)
                    cu_v[sl] = jnp.zeros((_LANES,), jnp.uint32)
                    cids_v[sl] = jnp.zeros((_LANES,), jnp.int32)
                    cpos_v[sl] = jnp.full((_LANES,), 4095, jnp.int32)
                    return carry

                lax.fori_loop(0, init_steps, init, 0)

                def step(i, off):
                    sl = pl.ds(i * _LANES, _LANES)
                    kv = u_v[sl]
                    mask = kv >= b_vec

                    @pl.when(off <= _CAND - _LANES)
                    def _store():
                        dst = pl.ds(off, _LANES)
                        pos = i * _LANES + lax.iota(jnp.int32, _LANES)
                        plsc.store_compressed(cu_v.at[dst], kv, mask=mask)
                        plsc.store_compressed(cids_v.at[dst], ids_v[sl],
                                              mask=mask)
                        plsc.store_compressed(cpos_v.at[dst], pos, mask=mask)

                    return off + jnp.sum(mask.astype(jnp.int32))

                lax.fori_loop(0, chunks, step, jnp.int32(0))
                pltpu.sync_copy(cu_v, cu_hbm.at[cls])
                pltpu.sync_copy(cids_v, cids_hbm.at[cls])
                pltpu.sync_copy(cpos_v, cpos_hbm.at[cls])

    return compact_kernel(u_keys, ids, thresh)


def _tc_rank_emit(cu, cids, cpos):
    c = cu.shape[0]
    n = _CAND
    kp = _TARGET

    def body(ur_ref, pr_ref, uc_ref, pc_ref, ic_ref, out_ref):
        u_row = ur_ref[0]
        p_row = pr_ref[0]
        u_col = uc_ref[0]
        p_col = pc_ref[0]
        beats = (u_row > u_col) | ((u_row == u_col) & (p_row < p_col))
        rank = jnp.sum(beats.astype(jnp.int32), axis=1, keepdims=True)
        rr = lax.broadcasted_iota(jnp.int32, (1, kp), 1)
        sel = jnp.sum(jnp.where(rank == rr, ic_ref[0], 0), axis=0,
                      keepdims=True)
        out_ref[...] = sel[None]

    out = pl.pallas_call(
        body,
        grid=(c,),
        in_specs=[
            pl.BlockSpec((1, 1, n), lambda i: (i, 0, 0)),
            pl.BlockSpec((1, 1, n), lambda i: (i, 0, 0)),
            pl.BlockSpec((1, n, 1), lambda i: (i, 0, 0)),
            pl.BlockSpec((1, n, 1), lambda i: (i, 0, 0)),
            pl.BlockSpec((1, n, 1), lambda i: (i, 0, 0)),
        ],
        out_specs=pl.BlockSpec((1, 1, kp), lambda i: (i, 0, 0)),
        out_shape=jax.ShapeDtypeStruct((c, 1, kp), jnp.int32),
    )(
        cu.reshape(c, 1, n),
        cpos.reshape(c, 1, n),
        cu.reshape(c, n, 1),
        cpos.reshape(c, n, 1),
        cids.reshape(c, n, 1),
    )
    return out.reshape(c, kp)


def kernel(in_degrees, ids_per_cls, budget):
    c, m = ids_per_cls.shape
    k = min(500, m)
    ids = ids_per_cls.astype(jnp.int32)

    gumbel = jnp.asarray(
        jax.random.gumbel(jax.random.key(42), (c, m), dtype=jnp.float32))

    n = c * m
    n_pad = -(-n // 512) * 512
    flat_ids = jnp.concatenate(
        [ids.reshape(-1), jnp.zeros((n_pad - n,), jnp.int32)])
    w = _sc_gather(in_degrees.astype(jnp.float32), flat_ids)[:n].reshape(c, m)
    w16 = w.astype(jnp.float16).astype(jnp.float32)

    u_keys, thresh = _tc_keys_threshold(w16, gumbel)

    cu, cids, cpos = _sc_compact(u_keys, ids, thresh)

    sel = _tc_rank_emit(cu, cids, cpos)
    return sel[:, :k].reshape(-1).astype(ids_per_cls.dtype)


# speedup vs baseline: 4.1544x; 1.0026x over previous
import functools

import jax
import jax.numpy as jnp
from jax import lax
from jax.experimental import pallas as pl
from jax.experimental.pallas import tpu as pltpu
from jax.experimental.pallas import tpu_sc as plsc

_LANES = 16
_TARGET = 500
_CAND = 528


def _sc_gather(table, ids):
    n_table = table.shape[0]
    c, m = ids.shape
    steps = m // _LANES
    info = plsc.get_sparse_core_info()
    n_workers = info.num_cores * info.num_subcores
    n_rounds = -(-c // n_workers)
    mesh = plsc.VectorSubcoreMesh(core_axis_name="c", subcore_axis_name="s")

    @functools.partial(
        pl.kernel,
        mesh=mesh,
        compiler_params=pltpu.CompilerParams(needs_layout_passes=False),
        out_type=jax.ShapeDtypeStruct((c, m), jnp.float32),
        scratch_types=[
            pltpu.VMEM((n_table,), jnp.float32),
            pltpu.VMEM((m,), jnp.int32),
            pltpu.VMEM((m,), jnp.float32),
        ],
    )
    def gather_kernel(deg_hbm, ids_hbm, out_hbm, table_v, idx_v, out_v):
        wid = lax.axis_index("s") * info.num_cores + lax.axis_index("c")
        pltpu.sync_copy(deg_hbm, table_v)

        for rnd in range(n_rounds):
            cls = wid + rnd * n_workers

            @pl.when(cls < c)
            def _process():
                pltpu.sync_copy(ids_hbm.at[cls], idx_v)

                def body(i, carry):
                    off = i * _LANES
                    idx = idx_v[pl.ds(off, _LANES)]
                    out_v[pl.ds(off, _LANES)] = plsc.load_gather(
                        table_v, [idx])
                    return carry

                lax.fori_loop(0, steps, body, 0)
                pltpu.sync_copy(out_v, out_hbm.at[cls])

    return gather_kernel(table, ids)


def _tc_keys_threshold(w16, gumbel):
    c, m = w16.shape

    def body(w_ref, g_ref, u_ref, b_ref):
        s = jnp.log(w_ref[...] + jnp.float32(1e-30)) + g_ref[...]
        b = lax.bitcast_convert_type(s, jnp.uint32)
        neg = (b >> jnp.uint32(31)) == jnp.uint32(1)
        u = b ^ jnp.where(neg, jnp.uint32(0xFFFFFFFF), jnp.uint32(0x80000000))
        u_ref[...] = u

        def bit_step(i, acc):
            bit = jnp.uint32(1) << (jnp.uint32(31) - i.astype(jnp.uint32))
            cand = acc | bit
            cnt = jnp.sum((u >= cand).astype(jnp.int32), axis=1,
                          keepdims=True)
            return jnp.where(cnt >= _TARGET, cand, acc)

        bsel = lax.fori_loop(0, 32, bit_step, jnp.zeros((c, 1), jnp.uint32))
        b_ref[...] = jnp.broadcast_to(bsel, (c, _LANES))

    return pl.pallas_call(
        body,
        out_shape=(
            jax.ShapeDtypeStruct((c, m), jnp.uint32),
            jax.ShapeDtypeStruct((c, _LANES), jnp.uint32),
        ),
    )(w16, gumbel)


def _sc_compact(u_keys, ids, thresh):
    c, m = u_keys.shape
    chunks = m // _LANES
    init_steps = _CAND // _LANES
    info = plsc.get_sparse_core_info()
    n_workers = info.num_cores * info.num_subcores
    n_rounds = -(-c // n_workers)
    mesh = plsc.VectorSubcoreMesh(core_axis_name="c", subcore_axis_name="s")

    @functools.partial(
        pl.kernel,
        mesh=mesh,
        compiler_params=pltpu.CompilerParams(needs_layout_passes=False),
        out_type=(
            jax.ShapeDtypeStruct((c, _CAND), jnp.uint32),
            jax.ShapeDtypeStruct((c, _CAND), jnp.int32),
            jax.ShapeDtypeStruct((c, _CAND), jnp.int32),
        ),
        scratch_types=[
            pltpu.VMEM((m,), jnp.uint32),
            pltpu.VMEM((m,), jnp.int32),
            pltpu.VMEM((_LANES,), jnp.uint32),
            pltpu.VMEM((_CAND,), jnp.uint32),
            pltpu.VMEM((_CAND,), jnp.int32),
            pltpu.VMEM((_CAND,), jnp.int32),
        ],
    )
    def compact_kernel(u_hbm, ids_hbm, b_hbm, cu_hbm, cids_hbm, cpos_hbm,
                       u_v, ids_v, b_v, cu_v, cids_v, cpos_v):
        wid = lax.axis_index("s") * info.num_cores + lax.axis_index("c")

        for rnd in range(n_rounds):
            cls = wid + rnd * n_workers

            @pl.when(cls < c)
            def _process():
                pltpu.sync_copy(u_hbm.at[cls], u_v)
                pltpu.sync_copy(ids_hbm.at[cls], ids_v)
                pltpu.sync_copy(b_hbm.at[cls], b_v)
                b_vec = b_v[...]

                def init(i, carry):
                    sl = pl.ds(i * _LANES, _LANES)
                    cu_v[sl] = jnp.zeros((_LANES,), jnp.uint32)
                    cids_v[sl] = jnp.zeros((_LANES,), jnp.int32)
                    cpos_v[sl] = jnp.full((_LANES,), 4095, jnp.int32)
                    return carry

                lax.fori_loop(0, init_steps, init, 0)

                def step(i, off):
                    sl = pl.ds(i * _LANES, _LANES)
                    kv = u_v[sl]
                    mask = kv >= b_vec

                    @pl.when(off <= _CAND - _LANES)
                    def _store():
                        dst = pl.ds(off, _LANES)
                        pos = i * _LANES + lax.iota(jnp.int32, _LANES)
                        plsc.store_compressed(cu_v.at[dst], kv, mask=mask)
                        plsc.store_compressed(cids_v.at[dst], ids_v[sl],
                                              mask=mask)
                        plsc.store_compressed(cpos_v.at[dst], pos, mask=mask)

                    return off + jnp.sum(mask.astype(jnp.int32))

                lax.fori_loop(0, chunks, step, jnp.int32(0))
                pltpu.sync_copy(cu_v, cu_hbm.at[cls])
                pltpu.sync_copy(cids_v, cids_hbm.at[cls])
                pltpu.sync_copy(cpos_v, cpos_hbm.at[cls])

    return compact_kernel(u_keys, ids, thresh)


def _tc_rank_emit(cu, cids, cpos):
    c = cu.shape[0]
    n = _CAND
    kp = _TARGET

    def body(ur_ref, pr_ref, uc_ref, pc_ref, ic_ref, out_ref):
        rr = lax.broadcasted_iota(jnp.int32, (1, kp), 1)

        def cls_body(ci, carry):
            sl = pl.ds(ci, 1)
            u_row = ur_ref[sl].reshape(1, n)
            p_row = pr_ref[sl].reshape(1, n)
            u_col = uc_ref[sl].reshape(n, 1)
            p_col = pc_ref[sl].reshape(n, 1)
            beats = (u_row > u_col) | ((u_row == u_col) & (p_row < p_col))
            rank = jnp.sum(beats.astype(jnp.int32), axis=1, keepdims=True)
            ids_col = ic_ref[sl].reshape(n, 1)
            sel = jnp.sum(jnp.where(rank == rr, ids_col, 0), axis=0,
                          keepdims=True)
            out_ref[sl] = sel[None]
            return carry

        lax.fori_loop(0, c, cls_body, 0)

    out = pl.pallas_call(
        body,
        out_shape=jax.ShapeDtypeStruct((c, 1, kp), jnp.int32),
    )(
        cu.reshape(c, 1, n),
        cpos.reshape(c, 1, n),
        cu.reshape(c, n, 1),
        cpos.reshape(c, n, 1),
        cids.reshape(c, n, 1),
    )
    return out.reshape(c, kp)


def kernel(in_degrees, ids_per_cls, budget):
    c, m = ids_per_cls.shape
    k = min(500, m)
    ids = ids_per_cls.astype(jnp.int32)

    gumbel = jnp.asarray(
        jax.random.gumbel(jax.random.key(42), (c, m), dtype=jnp.float32))

    w = _sc_gather(in_degrees.astype(jnp.float32), ids)
    w16 = w.astype(jnp.float16).astype(jnp.float32)

    u_keys, thresh = _tc_keys_threshold(w16, gumbel)

    cu, cids, cpos = _sc_compact(u_keys, ids, thresh)

    sel = _tc_rank_emit(cu, cids, cpos)
    return sel[:, :k].reshape(-1).astype(ids_per_cls.dtype)


# speedup vs baseline: 5.1478x; 1.2391x over previous
import functools

import jax
import jax.numpy as jnp
from jax import lax
from jax.experimental import pallas as pl
from jax.experimental.pallas import tpu as pltpu
from jax.experimental.pallas import tpu_sc as plsc

_LANES = 16
_TARGET = 500
_CAND = 528


def _sc_gather(table, ids):
    n_table = table.shape[0]
    c, m = ids.shape
    steps = m // _LANES
    info = plsc.get_sparse_core_info()
    n_workers = info.num_cores * info.num_subcores
    n_rounds = -(-c // n_workers)
    mesh = plsc.VectorSubcoreMesh(core_axis_name="c", subcore_axis_name="s")

    @functools.partial(
        pl.kernel,
        mesh=mesh,
        compiler_params=pltpu.CompilerParams(needs_layout_passes=False),
        out_type=jax.ShapeDtypeStruct((c, m), jnp.float32),
        scratch_types=[
            pltpu.VMEM((n_table,), jnp.float32),
            pltpu.VMEM((m,), jnp.int32),
            pltpu.VMEM((m,), jnp.float32),
        ],
    )
    def gather_kernel(deg_hbm, ids_hbm, out_hbm, table_v, idx_v, out_v):
        wid = lax.axis_index("s") * info.num_cores + lax.axis_index("c")
        pltpu.sync_copy(deg_hbm, table_v)

        for rnd in range(n_rounds):
            cls = wid + rnd * n_workers

            @pl.when(cls < c)
            def _process():
                pltpu.sync_copy(ids_hbm.at[cls], idx_v)

                def body(i, carry):
                    off = i * _LANES
                    idx = idx_v[pl.ds(off, _LANES)]
                    out_v[pl.ds(off, _LANES)] = plsc.load_gather(
                        table_v, [idx])
                    return carry

                lax.fori_loop(0, steps, body, 0)
                pltpu.sync_copy(out_v, out_hbm.at[cls])

    return gather_kernel(table, ids)


def _tc_keys_threshold(w16, gumbel):
    c, m = w16.shape

    def body(w_ref, g_ref, u_ref, b_ref):
        s = jnp.log(w_ref[...] + jnp.float32(1e-30)) + g_ref[...]
        b = lax.bitcast_convert_type(s, jnp.uint32)
        neg = (b >> jnp.uint32(31)) == jnp.uint32(1)
        u = b ^ jnp.where(neg, jnp.uint32(0xFFFFFFFF), jnp.uint32(0x80000000))
        u_ref[...] = u

        def bit_step(i, acc):
            bit = jnp.uint32(1) << (jnp.uint32(31) - i.astype(jnp.uint32))
            cand = acc | bit
            cnt = jnp.sum((u >= cand).astype(jnp.int32), axis=1,
                          keepdims=True)
            return jnp.where(cnt >= _TARGET, cand, acc)

        bsel = lax.fori_loop(0, 32, bit_step, jnp.zeros((c, 1), jnp.uint32))
        b_ref[...] = jnp.broadcast_to(bsel, (c, _LANES))

    return pl.pallas_call(
        body,
        out_shape=(
            jax.ShapeDtypeStruct((c, m), jnp.uint32),
            jax.ShapeDtypeStruct((c, _LANES), jnp.uint32),
        ),
    )(w16, gumbel)


def _sc_compact(u_keys, ids, thresh):
    c, m = u_keys.shape
    chunks = m // _LANES
    init_steps = _CAND // _LANES
    info = plsc.get_sparse_core_info()
    n_workers = info.num_cores * info.num_subcores
    n_rounds = -(-c // n_workers)
    mesh = plsc.VectorSubcoreMesh(core_axis_name="c", subcore_axis_name="s")

    @functools.partial(
        pl.kernel,
        mesh=mesh,
        compiler_params=pltpu.CompilerParams(needs_layout_passes=False),
        out_type=(
            jax.ShapeDtypeStruct((c, _CAND), jnp.uint32),
            jax.ShapeDtypeStruct((c, _CAND), jnp.int32),
        ),
        scratch_types=[
            pltpu.VMEM((m,), jnp.uint32),
            pltpu.VMEM((m,), jnp.int32),
            pltpu.VMEM((_LANES,), jnp.uint32),
            pltpu.VMEM((_CAND,), jnp.uint32),
            pltpu.VMEM((_CAND,), jnp.int32),
        ],
    )
    def compact_kernel(u_hbm, ids_hbm, b_hbm, cu_hbm, cids_hbm,
                       u_v, ids_v, b_v, cu_v, cids_v):
        wid = lax.axis_index("s") * info.num_cores + lax.axis_index("c")

        for rnd in range(n_rounds):
            cls = wid + rnd * n_workers

            @pl.when(cls < c)
            def _process():
                pltpu.sync_copy(u_hbm.at[cls], u_v)
                pltpu.sync_copy(ids_hbm.at[cls], ids_v)
                pltpu.sync_copy(b_hbm.at[cls], b_v)
                b_vec = b_v[...]

                def init(i, carry):
                    sl = pl.ds(i * _LANES, _LANES)
                    cu_v[sl] = jnp.zeros((_LANES,), jnp.uint32)
                    cids_v[sl] = jnp.zeros((_LANES,), jnp.int32)
                    return carry

                lax.fori_loop(0, init_steps, init, 0)

                def step(i, off):
                    sl = pl.ds(i * _LANES, _LANES)
                    kv = u_v[sl]
                    mask = kv >= b_vec

                    @pl.when(off <= _CAND - _LANES)
                    def _store():
                        dst = pl.ds(off, _LANES)
                        plsc.store_compressed(cu_v.at[dst], kv, mask=mask)
                        plsc.store_compressed(cids_v.at[dst], ids_v[sl],
                                              mask=mask)

                    return off + jnp.sum(mask.astype(jnp.int32))

                lax.fori_loop(0, chunks, step, jnp.int32(0))
                pltpu.sync_copy(cu_v, cu_hbm.at[cls])
                pltpu.sync_copy(cids_v, cids_hbm.at[cls])

    return compact_kernel(u_keys, ids, thresh)


def _tc_rank_emit(cu, cids):
    c = cu.shape[0]
    n = _CAND
    kp = _TARGET

    def body(ur_ref, uc_ref, ic_ref, out_ref):
        rr = lax.broadcasted_iota(jnp.int32, (1, kp), 1)
        jj = lax.broadcasted_iota(jnp.int32, (1, n), 1)
        ii = lax.broadcasted_iota(jnp.int32, (n, 1), 0)
        tri = jj < ii

        def cls_body(ci, carry):
            sl = pl.ds(ci, 1)
            u_row = ur_ref[sl].reshape(1, n)
            u_col = uc_ref[sl].reshape(n, 1)
            beats = (u_row > u_col) | ((u_row == u_col) & tri)
            rank = jnp.sum(beats.astype(jnp.int32), axis=1, keepdims=True)
            ids_col = ic_ref[sl].reshape(n, 1)
            sel = jnp.sum(jnp.where(rank == rr, ids_col, 0), axis=0,
                          keepdims=True)
            out_ref[sl] = sel[None]
            return carry

        lax.fori_loop(0, c, cls_body, 0)

    out = pl.pallas_call(
        body,
        out_shape=jax.ShapeDtypeStruct((c, 1, kp), jnp.int32),
    )(
        cu.reshape(c, 1, n),
        cu.reshape(c, n, 1),
        cids.reshape(c, n, 1),
    )
    return out.reshape(c, kp)


def kernel(in_degrees, ids_per_cls, budget):
    c, m = ids_per_cls.shape
    k = min(500, m)
    ids = ids_per_cls.astype(jnp.int32)

    gumbel = jnp.asarray(
        jax.random.gumbel(jax.random.key(42), (c, m), dtype=jnp.float32))

    w = _sc_gather(in_degrees.astype(jnp.float32), ids)
    w16 = w.astype(jnp.float16).astype(jnp.float32)

    u_keys, thresh = _tc_keys_threshold(w16, gumbel)

    cu, cids = _sc_compact(u_keys, ids, thresh)

    sel = _tc_rank_emit(cu, cids)
    return sel[:, :k].reshape(-1).astype(ids_per_cls.dtype)
